# Initial kernel scaffold; baseline (speedup 1.0000x reference)
import jax
import jax.numpy as jnp
from jax.experimental import pallas as pl


def kernel(des, tweet, num_prop, cat_prop, edge_index, edge_type, W_t, b_t, W_in, b_in, W_rel, W_root, b_rgcn, W_o1, b_o1, W_o2, b_o2):
    raise NotImplementedError("write your pallas kernel here")


# speedup vs baseline: 5.2393x; 5.2393x over previous
import functools

import jax
import jax.numpy as jnp
from jax import lax
from jax.experimental import pallas as pl
from jax.experimental.pallas import tpu as pltpu
from jax.experimental.pallas import tpu_sc as plsc

N = 10000
E = 320000
R = 5
D = 128
TS = 768

NC = 2
NS = 16
LANES = 16

CHUNK = 16
NCHUNK = D // CHUNK
STEP = 128
NSTEPS = E // STEP
RN = R * N
RN_PAD = 50176
CNT_PAD = 50176
BN = 400


def _leaky(x):
    return jnp.where(x >= 0, x, 0.01 * x)


def _mesh():
    return plsc.VectorSubcoreMesh(
        core_axis_name="c", subcore_axis_name="s",
        num_cores=NC, num_subcores=NS)


def _sc_index_body(src_hbm, dst_hbm, et_hbm, idx_pack, cnt_out,
                   sbuf, dbuf, ebuf, idxout, ones_v, zbuf, cbuf, cnt_sp):
    core = lax.axis_index("c")
    sub = lax.axis_index("s")
    gid = core * NS + sub

    for j in range(STEP // LANES):
        ones_v[pl.ds(j * LANES, LANES)] = jnp.full((LANES,), 1.0, jnp.float32)
    for j in range(448 // LANES):
        zbuf[pl.ds(j * LANES, LANES)] = jnp.zeros((LANES,), jnp.float32)

    tile_cnt = CNT_PAD // NS
    for j in range(tile_cnt // 448):
        pltpu.sync_copy(zbuf, cnt_sp.at[pl.ds(sub * tile_cnt + j * 448, 448)])
    plsc.subcore_barrier()

    base = gid * (NSTEPS // (NC * NS)) + jnp.minimum(gid, NSTEPS % (NC * NS))
    nsteps = NSTEPS // (NC * NS) + jnp.where(gid < NSTEPS % (NC * NS), 1, 0)

    def step_body(i, _):
        off = i * STEP
        pltpu.sync_copy(src_hbm.at[pl.ds(off, STEP)], sbuf)
        pltpu.sync_copy(dst_hbm.at[pl.ds(off, STEP)], dbuf)
        pltpu.sync_copy(et_hbm.at[pl.ds(off, STEP)], ebuf)
        for j in range(STEP // LANES):
            sl = pl.ds(j * LANES, LANES)
            en = ebuf[sl] * N
            idxout[0, sl] = (en + sbuf[sl]) * NCHUNK
            idxout[1, sl] = en + dbuf[sl]
        pltpu.sync_copy(idxout, idx_pack.at[i])
        pltpu.sync_copy(ones_v, cnt_sp.at[idxout.at[1]], add=True)
        return _

    lax.fori_loop(base, base + nsteps, step_body, None)
    plsc.subcore_barrier()

    pltpu.sync_copy(cnt_sp.at[pl.ds(sub * tile_cnt, tile_cnt)], cbuf)
    pltpu.sync_copy(cbuf, cnt_out.at[pl.ds(core * CNT_PAD + sub * tile_cnt,
                                           tile_cnt)])


def _sc_index(src, dst, et):
    f = pl.kernel(
        _sc_index_body,
        out_type=[
            jax.ShapeDtypeStruct((NSTEPS, 2, STEP), jnp.int32),
            jax.ShapeDtypeStruct((NC * CNT_PAD,), jnp.float32),
        ],
        mesh=_mesh(),
        compiler_params=pltpu.CompilerParams(use_tc_tiling_on_sc=False),
        scratch_types=[
            pltpu.VMEM((STEP,), jnp.int32),
            pltpu.VMEM((STEP,), jnp.int32),
            pltpu.VMEM((STEP,), jnp.int32),
            pltpu.VMEM((2, STEP), jnp.int32),
            pltpu.VMEM((STEP,), jnp.float32),
            pltpu.VMEM((448,), jnp.float32),
            pltpu.VMEM((CNT_PAD // NS,), jnp.float32),
            pltpu.VMEM_SHARED((CNT_PAD,), jnp.float32),
        ],
    )
    return f(src, dst, et)


def _sc_scatter_body(hflat, idx_pack, t_out, idxbuf, gbuf, rows, zrows, tbuf,
                     t_sp):
    core = lax.axis_index("c")
    sub = lax.axis_index("s")

    ZR = 112
    for j in range(ZR):
        for k in range(CHUNK // LANES):
            zrows[j, pl.ds(k * LANES, LANES)] = jnp.zeros((LANES,),
                                                          jnp.float32)

    tile_rows = RN_PAD // NS
    base = sub * (NSTEPS // NS) + jnp.minimum(sub, NSTEPS % NS)
    nsteps = NSTEPS // NS + jnp.where(sub < NSTEPS % NS, 1, 0)

    for p in range(NCHUNK // NC):
        chunk = core * (NCHUNK // NC) + p

        for j in range(tile_rows // ZR):
            pltpu.sync_copy(
                zrows, t_sp.at[pl.ds(sub * tile_rows + j * ZR, ZR)])
        plsc.subcore_barrier()

        def step_body(i, _):
            pltpu.sync_copy(idx_pack.at[i], idxbuf)
            for j in range(STEP // LANES):
                sl = pl.ds(j * LANES, LANES)
                gbuf[sl] = idxbuf[0, sl] + chunk
            pltpu.sync_copy(hflat.at[gbuf], rows)
            pltpu.sync_copy(rows, t_sp.at[idxbuf.at[1]], add=True)
            return _

        lax.fori_loop(base, base + nsteps, step_body, None)
        plsc.subcore_barrier()

        for j in range(tile_rows // 784):
            off = sub * tile_rows + j * 784
            pltpu.sync_copy(t_sp.at[pl.ds(off, 784)], tbuf)
            pltpu.sync_copy(tbuf, t_out.at[chunk, pl.ds(off, 784)])
        plsc.subcore_barrier()


def _sc_scatter(hflat, idx_pack):
    f = pl.kernel(
        _sc_scatter_body,
        out_type=[jax.ShapeDtypeStruct((NCHUNK, RN_PAD, CHUNK), jnp.float32)],
        mesh=_mesh(),
        compiler_params=pltpu.CompilerParams(use_tc_tiling_on_sc=False),
        scratch_types=[
            pltpu.VMEM((2, STEP), jnp.int32),
            pltpu.VMEM((STEP,), jnp.int32),
            pltpu.VMEM((STEP, CHUNK), jnp.float32),
            pltpu.VMEM((112, CHUNK), jnp.float32),
            pltpu.VMEM((784, CHUNK), jnp.float32),
            pltpu.VMEM_SHARED((RN_PAD, CHUNK), jnp.float32),
        ],
    )
    return f(hflat, idx_pack)[0]


def _tc_encode_body(tw_ref, wt_ref, bt_ref, win_ref, bin_ref, wrel_ref,
                    x_ref, h_ref):
    t = _leaky(jnp.dot(tw_ref[...], wt_ref[...],
                       preferred_element_type=jnp.float32) + bt_ref[...])
    x = _leaky(jnp.dot(t, win_ref[...],
                       preferred_element_type=jnp.float32) + bin_ref[...])
    x_ref[...] = x
    for r in range(R):
        h_ref[r] = jnp.dot(x, wrel_ref[r], preferred_element_type=jnp.float32)


def _tc_encode(tweet, W_t, b_t, W_in, b_in, W_rel):
    return pl.pallas_call(
        _tc_encode_body,
        grid=(N // BN,),
        in_specs=[
            pl.BlockSpec((BN, TS), lambda i: (i, 0)),
            pl.BlockSpec((TS, D), lambda i: (0, 0)),
            pl.BlockSpec((1, D), lambda i: (0, 0)),
            pl.BlockSpec((D, D), lambda i: (0, 0)),
            pl.BlockSpec((1, D), lambda i: (0, 0)),
            pl.BlockSpec((R, D, D), lambda i: (0, 0, 0)),
        ],
        out_specs=[
            pl.BlockSpec((BN, D), lambda i: (i, 0)),
            pl.BlockSpec((R, BN, D), lambda i: (0, i, 0)),
        ],
        out_shape=[
            jax.ShapeDtypeStruct((N, D), jnp.float32),
            jax.ShapeDtypeStruct((R, N, D), jnp.float32),
        ],
    )(tweet, W_t, b_t.reshape(1, D), W_in, b_in.reshape(1, D), W_rel)


def _combine(x_ref, t_ref, cnt_ref, wroot_ref, brg_ref):
    base = jnp.dot(x_ref[...], wroot_ref[...],
                   preferred_element_type=jnp.float32) + brg_ref[...]
    nrms = [1.0 / jnp.maximum(cnt_ref[0, r] + cnt_ref[1, r], 1.0)
            for r in range(R)]
    cols = []
    for c in range(NCHUNK):
        col = t_ref[c, 0] * nrms[0]
        for r in range(1, R):
            col = col + t_ref[c, r] * nrms[r]
        cols.append(base[:, c * CHUNK:(c + 1) * CHUNK] + col)
    return jnp.concatenate(cols, axis=1)


def _tc_combine1_body(x_ref, t_ref, cnt_ref, wroot_ref, brg_ref, wrel_ref,
                      agg_ref, h_ref):
    agg = _combine(x_ref, t_ref, cnt_ref, wroot_ref, brg_ref)
    agg_ref[...] = agg
    for r in range(R):
        h_ref[r] = jnp.dot(agg, wrel_ref[r],
                           preferred_element_type=jnp.float32)


def _tc_combine1(x, t4, cnt4, W_root, b_rgcn, W_rel):
    return pl.pallas_call(
        _tc_combine1_body,
        grid=(N // BN,),
        in_specs=[
            pl.BlockSpec((BN, D), lambda i: (i, 0)),
            pl.BlockSpec((NCHUNK, R, BN, CHUNK), lambda i: (0, 0, i, 0)),
            pl.BlockSpec((NC, R, BN, 1), lambda i: (0, 0, i, 0)),
            pl.BlockSpec((D, D), lambda i: (0, 0)),
            pl.BlockSpec((1, D), lambda i: (0, 0)),
            pl.BlockSpec((R, D, D), lambda i: (0, 0, 0)),
        ],
        out_specs=[
            pl.BlockSpec((BN, D), lambda i: (i, 0)),
            pl.BlockSpec((R, BN, D), lambda i: (0, i, 0)),
        ],
        out_shape=[
            jax.ShapeDtypeStruct((N, D), jnp.float32),
            jax.ShapeDtypeStruct((R, N, D), jnp.float32),
        ],
    )(x, t4, cnt4, W_root, b_rgcn.reshape(1, D), W_rel)


def _tc_combine2_body(x_ref, t_ref, cnt_ref, wroot_ref, brg_ref,
                      wo1_ref, bo1_ref, wo2_ref, bo2_ref, out_ref):
    agg = _combine(x_ref, t_ref, cnt_ref, wroot_ref, brg_ref)
    o = _leaky(jnp.dot(agg, wo1_ref[...],
                       preferred_element_type=jnp.float32) + bo1_ref[...])
    out_ref[...] = jnp.dot(o, wo2_ref[...],
                           preferred_element_type=jnp.float32) + bo2_ref[...]


def _tc_combine2(x, t4, cnt4, W_root, b_rgcn, W_o1, b_o1, W_o2p, b_o2p):
    return pl.pallas_call(
        _tc_combine2_body,
        grid=(N // BN,),
        in_specs=[
            pl.BlockSpec((BN, D), lambda i: (i, 0)),
            pl.BlockSpec((NCHUNK, R, BN, CHUNK), lambda i: (0, 0, i, 0)),
            pl.BlockSpec((NC, R, BN, 1), lambda i: (0, 0, i, 0)),
            pl.BlockSpec((D, D), lambda i: (0, 0)),
            pl.BlockSpec((1, D), lambda i: (0, 0)),
            pl.BlockSpec((D, D), lambda i: (0, 0)),
            pl.BlockSpec((1, D), lambda i: (0, 0)),
            pl.BlockSpec((D, D), lambda i: (0, 0)),
            pl.BlockSpec((1, D), lambda i: (0, 0)),
        ],
        out_specs=pl.BlockSpec((BN, D), lambda i: (i, 0)),
        out_shape=jax.ShapeDtypeStruct((N, D), jnp.float32),
    )(x, t4, cnt4, W_root, b_rgcn.reshape(1, D),
      W_o1, b_o1.reshape(1, D), W_o2p, b_o2p)


def kernel(des, tweet, num_prop, cat_prop, edge_index, edge_type,
           W_t, b_t, W_in, b_in, W_rel, W_root, b_rgcn, W_o1, b_o1,
           W_o2, b_o2):
    src = edge_index[0]
    dst = edge_index[1]
    et = edge_type.astype(jnp.int32)

    idx_pack, cnt = _sc_index(src, dst, et)
    cnt4 = cnt.reshape(NC, CNT_PAD)[:, :RN].reshape(NC, R, N, 1)

    x1, h1 = _tc_encode(tweet, W_t, b_t, W_in, b_in, W_rel)

    t1 = _sc_scatter(h1.reshape(RN * NCHUNK, CHUNK), idx_pack)
    agg1, h2 = _tc_combine1(x1, t1[:, :RN].reshape(NCHUNK, R, N, CHUNK), cnt4,
                            W_root, b_rgcn, W_rel)

    t2 = _sc_scatter(h2.reshape(RN * NCHUNK, CHUNK), idx_pack)

    W_o2p = jnp.pad(W_o2, ((0, 0), (0, D - 2)))
    b_o2p = jnp.pad(b_o2, (0, D - 2)).reshape(1, D)
    out = _tc_combine2(agg1, t2[:, :RN].reshape(NCHUNK, R, N, CHUNK), cnt4,
                       W_root, b_rgcn, W_o1, b_o1, W_o2p, b_o2p)
    return out[:, :2]
